# Initial kernel scaffold; baseline (speedup 1.0000x reference)
import jax
import jax.numpy as jnp
from jax.experimental import pallas as pl


def kernel(feat, edge_index, W, b):
    raise NotImplementedError("write your pallas kernel here")


# speedup vs baseline: 4.0994x; 4.0994x over previous
import functools

import jax
import jax.numpy as jnp
from jax import lax
from jax.experimental import pallas as pl
from jax.experimental.pallas import tpu as pltpu
from jax.experimental.pallas import tpu_sc as plsc

N_NODES = 10000
N_EDGES = 320000
D = 128
NC = 2
NS = 16
NW = NC * NS
N_PAD = 10240
E_PAD = 323584
CH = 128
EPW = E_PAD // NW
CPW = EPW // CH
RPT = N_PAD // NS
CW = 16

_F32 = jnp.float32


def _mesh():
    return plsc.VectorSubcoreMesh(
        core_axis_name="c", subcore_axis_name="s",
        num_cores=NC, num_subcores=NS)


@functools.partial(
    pl.kernel,
    out_type=jax.ShapeDtypeStruct((NW, 2, N_PAD), _F32),
    mesh=_mesh(),
    scratch_types=[
        pltpu.VMEM((CH,), jnp.int32),
        pltpu.VMEM((CH,), jnp.int32),
        pltpu.VMEM((N_PAD,), _F32),
        pltpu.VMEM((N_PAD,), _F32),
    ],
    compiler_params=pltpu.CompilerParams(needs_layout_passes=False),
)
def _deg_kernel(src_hbm, dst_hbm, out_hbm, sidx, didx, cs, cd):
    cid = lax.axis_index("c")
    sid = lax.axis_index("s")
    wid = sid * NC + cid
    z = jnp.zeros((16,), _F32)

    def zbody(i, carry):
        cs[pl.ds(i * 16, 16)] = z
        cd[pl.ds(i * 16, 16)] = z
        return carry

    lax.fori_loop(0, N_PAD // 16, zbody, 0)
    ones16 = jnp.ones((16,), _F32)

    def body(k, carry):
        base = wid * EPW + k * CH
        pltpu.sync_copy(src_hbm.at[pl.ds(base, CH)], sidx)
        pltpu.sync_copy(dst_hbm.at[pl.ds(base, CH)], didx)
        for j in range(CH // 16):
            sv = sidx[pl.ds(j * 16, 16)]
            dv = didx[pl.ds(j * 16, 16)]
            plsc.addupdate_scatter(cs, [sv], ones16)
            plsc.addupdate_scatter(cd, [dv], ones16)
        return carry

    lax.fori_loop(0, CPW, body, 0)
    pltpu.sync_copy(cs, out_hbm.at[wid, 0])
    pltpu.sync_copy(cd, out_hbm.at[wid, 1])


@functools.partial(
    pl.kernel,
    out_type=jax.ShapeDtypeStruct((NC, N_PAD, D), _F32),
    mesh=_mesh(),
    scratch_types=[
        pltpu.VMEM((CH,), jnp.int32),
        pltpu.VMEM((CH,), jnp.int32),
        pltpu.VMEM((CH, D), _F32),
        pltpu.VMEM_SHARED((N_PAD, D), _F32),
        pltpu.SemaphoreType.DMA,
    ],
)
def _agg_kernel(h_hbm, src_hbm, dst_hbm, zrows_hbm, out_hbm,
                sidx, didx, rows, acc_sh, sem):
    cid = lax.axis_index("c")
    sid = lax.axis_index("s")
    wid = sid * NC + cid
    pltpu.sync_copy(zrows_hbm, acc_sh.at[pl.ds(sid * RPT, RPT)])
    plsc.subcore_barrier()

    def body(k, carry):
        base = wid * EPW + k * CH
        pltpu.sync_copy(src_hbm.at[pl.ds(base, CH)], sidx)
        pltpu.async_copy(h_hbm.at[sidx], rows, sem).wait()
        pltpu.sync_copy(dst_hbm.at[pl.ds(base, CH)], didx)
        pltpu.sync_copy(rows, acc_sh.at[didx], add=True)
        return carry

    lax.fori_loop(0, CPW, body, 0)
    plsc.subcore_barrier()
    pltpu.sync_copy(acc_sh.at[pl.ds(sid * RPT, RPT)],
                    out_hbm.at[cid, pl.ds(sid * RPT, RPT)])


def _scale_body(cnt_ref, feat_ref, h_ref):
    deg = jnp.maximum(jnp.sum(cnt_ref[...], axis=1, keepdims=True), 1.0)
    h_ref[...] = feat_ref[...] * lax.rsqrt(deg)


_scale = pl.pallas_call(
    _scale_body,
    grid=(N_PAD // 1024,),
    in_specs=[pl.BlockSpec((1024, NW), lambda i: (i, 0)),
              pl.BlockSpec((1024, D), lambda i: (i, 0))],
    out_specs=pl.BlockSpec((1024, D), lambda i: (i, 0)),
    out_shape=jax.ShapeDtypeStruct((N_PAD, D), _F32),
)


def _final_body(a0_ref, a1_ref, cnt_ref, w_ref, b_ref, o_ref):
    a = a0_ref[...] + a1_ref[...]
    r = jnp.dot(a, w_ref[...], preferred_element_type=_F32)
    nd = lax.rsqrt(jnp.maximum(jnp.sum(cnt_ref[...], axis=1, keepdims=True), 1.0))
    o_ref[...] = r * nd + b_ref[...]


_final = pl.pallas_call(
    _final_body,
    grid=(N_PAD // 1024,),
    in_specs=[pl.BlockSpec((1024, D), lambda i: (i, 0)),
              pl.BlockSpec((1024, D), lambda i: (i, 0)),
              pl.BlockSpec((1024, NW), lambda i: (i, 0)),
              pl.BlockSpec((D, D), lambda i: (0, 0)),
              pl.BlockSpec((1, D), lambda i: (0, 0))],
    out_specs=pl.BlockSpec((1024, D), lambda i: (i, 0)),
    out_shape=jax.ShapeDtypeStruct((N_PAD, D), _F32),
)


def kernel(feat, edge_index, W, b):
    src = edge_index[0]
    dst = edge_index[1]
    pad = jnp.full((E_PAD - N_EDGES,), N_NODES, jnp.int32)
    src_p = jnp.concatenate([src, pad])
    dst_p = jnp.concatenate([dst, pad])
    feat_p = jnp.pad(feat, ((0, N_PAD - N_NODES), (0, 0)))

    counts = _deg_kernel(src_p, dst_p)

    cnt_s = counts[:, 0, :].T
    h = _scale(cnt_s, feat_p)

    zrows = jnp.zeros((RPT, D), _F32)
    acc = _agg_kernel(h, src_p, dst_p, zrows)

    cnt_d = counts[:, 1, :].T
    out = _final(acc[0], acc[1], cnt_d, W, b.reshape(1, D))
    return out[:N_NODES]
